# baseline (speedup 1.0000x reference)
import functools

import jax
import jax.numpy as jnp
from jax import lax
from jax.experimental import pallas as pl
from jax.experimental.pallas import tpu as pltpu
from jax.experimental.pallas import tpu_sc as plsc

_NUM_CORES = 2
_NUM_SUBCORES = 16
_NW = _NUM_CORES * _NUM_SUBCORES


def _sc_gather(means, question, idx):
    B = idx.shape[0]
    D = means.shape[1]
    b_per_w = B // _NW
    assert B % (8 * _NW) == 0

    mesh = plsc.VectorSubcoreMesh(core_axis_name="c", subcore_axis_name="s")

    @functools.partial(
        pl.kernel,
        out_type=(
            jax.ShapeDtypeStruct((B, D), jnp.float32),
            jax.ShapeDtypeStruct((B, D), jnp.float32),
        ),
        mesh=mesh,
        scratch_types=[
            pltpu.VMEM((b_per_w,), jnp.int32),
            pltpu.VMEM((b_per_w, D), jnp.float32),
            pltpu.VMEM((b_per_w, D), jnp.float32),
            pltpu.SemaphoreType.DMA,
            pltpu.SemaphoreType.DMA,
        ],
        compiler_params=pltpu.CompilerParams(use_tc_tiling_on_sc=False),
    )
    def gather_kernel(means_hbm, question_hbm, idx_hbm, gm_hbm, gq_hbm,
                      idx_v, m_v, q_v, sem_m, sem_q):
        wid = lax.axis_index("s") * _NUM_CORES + lax.axis_index("c")
        base = wid * b_per_w
        pltpu.sync_copy(idx_hbm.at[pl.ds(base, b_per_w)], idx_v)
        cm = pltpu.async_copy(means_hbm.at[idx_v], m_v, sem_m)
        cq = pltpu.async_copy(question_hbm.at[idx_v], q_v, sem_q)
        cm.wait()
        cq.wait()
        pltpu.sync_copy(m_v, gm_hbm.at[pl.ds(base, b_per_w)])
        pltpu.sync_copy(q_v, gq_hbm.at[pl.ds(base, b_per_w)])

    return gather_kernel(means, question, idx)


def _tc_subtract(pros3, gm, gq, block_b):
    B, R, L = pros3.shape

    def body(p_ref, m_ref, q_ref, o_ref):
        c = (m_ref[...] + q_ref[...]) * 0.5
        c2 = jnp.concatenate([c, c], axis=-1)
        o_ref[...] = p_ref[...] - c2[:, None, :]

    return pl.pallas_call(
        body,
        grid=(B // block_b,),
        in_specs=[
            pl.BlockSpec((block_b, R, L), lambda i: (i, 0, 0)),
            pl.BlockSpec((block_b, 64), lambda i: (i, 0)),
            pl.BlockSpec((block_b, 64), lambda i: (i, 0)),
        ],
        out_specs=pl.BlockSpec((block_b, R, L), lambda i: (i, 0, 0)),
        out_shape=jax.ShapeDtypeStruct((B, R, L), jnp.float32),
    )(pros3, gm, gq)


def kernel(prosody, spkr_id, means, question):
    B, T, D = prosody.shape
    idx = spkr_id.astype(jnp.int32)
    gm, gq = _sc_gather(means, question, idx)
    pros3 = prosody.reshape(B, T * D // 128, 128)
    out3 = _tc_subtract(pros3, gm, gq, block_b=256)
    return out3.reshape(B, T, D)
